# baseline (speedup 1.0000x reference)
import functools

import jax
import jax.numpy as jnp
from jax import lax
from jax.experimental import pallas as pl
from jax.experimental.pallas import tpu as pltpu
from jax.experimental.pallas import tpu_sc as plsc

E = 64
OUT_W = 2 * E + 1
PW = E // 2
TAB = 1000 * PW
L_DIM = 200
CH = 80
BLK_B = 4
BROWS = BLK_B * L_DIM
NCH = BROWS // CH


def _chunk_dsts(u):
    l = CH * u
    b_off, l0 = divmod(l, L_DIM)
    if l0 + CH <= L_DIM:
        return [(b_off, l0, CH, 0)]
    s1 = L_DIM - l0
    return [(b_off, l0, s1, 0), (b_off + 1, 0, CH - s1, s1)]


def _sc_body(bs_per_w, de1_hbm, de2_hbm, f_hbm, tab_hbm, out_hbm,
             idx1_v, idx2_v, f_v, tab_v, sb_0, sb_1, sem_o):
    nc = 2
    wid = lax.axis_index("s") * nc + lax.axis_index("c")
    base_b = wid * bs_per_w
    lanes = lax.iota(jnp.int32, 16)
    sbs = (sb_0, sb_1)
    n_blk = bs_per_w // BLK_B

    pltpu.sync_copy(tab_hbm, tab_v)

    def out_copies(b0, u, p):
        res = []
        for b_off, l0, sz, loc in _chunk_dsts(u):
            res.append(pltpu.make_async_copy(
                sbs[p].at[pl.ds(loc, sz)],
                out_hbm.at[b0 + b_off, pl.ds(l0, sz)], sem_o))
        return res

    def assemble(u, p):
        sb = sbs[p]
        mask16 = jnp.full((16,), -65536, jnp.int32)
        col128 = lanes * 0 + (2 * E)
        for g in range(CH // 16):
            rows = lanes + 16 * g
            off = CH * u + 16 * g
            i1 = idx1_v[pl.ds(off, 16)]
            i2 = idx2_v[pl.ds(off, 16)]
            fv = f_v[pl.ds(off, 16)]
            ga1_0 = i1 * PW
            ga2_0 = i2 * PW + TAB
            plsc.store_scatter(sb, [rows, col128], fv)

            def wave(cb, carry):
                a1, a2 = carry
                v1s = [plsc.load_gather(tab_v, [a1 + d]) for d in range(4)]
                v2s = [plsc.load_gather(tab_v, [a2 + d]) for d in range(4)]
                e1s, e2s = [], []
                for d in range(4):
                    e1s.append(plsc.bitcast(v1s[d] << 16, jnp.float32))
                    e1s.append(plsc.bitcast(v1s[d] & mask16, jnp.float32))
                    e2s.append(plsc.bitcast(v2s[d] << 16, jnp.float32))
                    e2s.append(plsc.bitcast(v2s[d] & mask16, jnp.float32))
                cbase = cb * 8
                for d in range(8):
                    plsc.store_scatter(sb, [rows, cbase + d + (lanes * 0)],
                                       e1s[d])
                for d in range(8):
                    plsc.store_scatter(sb, [rows, cbase + d + E + (lanes * 0)],
                                       e2s[d])
                return a1 + 4, a2 + 4

            lax.fori_loop(0, PW // 4, wave, (ga1_0, ga2_0))

    def blk_body(m, _):
        b0 = base_b + m * BLK_B
        r0 = pl.multiple_of(b0 * L_DIM, BROWS)
        pltpu.sync_copy(de1_hbm.at[pl.ds(r0, BROWS)], idx1_v)
        pltpu.sync_copy(de2_hbm.at[pl.ds(r0, BROWS)], idx2_v)
        pltpu.sync_copy(f_hbm.at[pl.ds(r0, BROWS)], f_v)
        for u in range(NCH):
            p = u % 2
            if u >= 2:
                for c in out_copies(b0, u - 2, p):
                    c.wait()
            else:
                @pl.when(m > 0)
                def _wait_prev():
                    for c in out_copies(b0 - BLK_B, NCH - 2 + u, p):
                        c.wait()
            assemble(u, p)
            for c in out_copies(b0, u, p):
                c.start()

    lax.fori_loop(0, n_blk, blk_body, None)
    b_last = base_b + (n_blk - 1) * BLK_B
    for u in (NCH - 2, NCH - 1):
        for c in out_copies(b_last, u, u % 2):
            c.wait()


def kernel(de1, de2, f, W1, W2):
    B, L = de1.shape
    n = B * L
    info = plsc.get_sparse_core_info()
    nw = info.num_cores * info.num_subcores
    bs_per_w = B // nw
    assert bs_per_w % BLK_B == 0 and L == L_DIM

    de1f = de1.reshape(n)
    de2f = de2.reshape(n)
    ff = f.reshape(n)
    p1 = lax.bitcast_convert_type(
        W1.astype(jnp.bfloat16).reshape(1000, PW, 2), jnp.int32)
    p2 = lax.bitcast_convert_type(
        W2.astype(jnp.bfloat16).reshape(1000, PW, 2), jnp.int32)
    tab = jnp.concatenate([p1.reshape(-1), p2.reshape(-1)])

    mesh = plsc.VectorSubcoreMesh(core_axis_name="c", subcore_axis_name="s")
    run = pl.kernel(
        functools.partial(_sc_body, bs_per_w),
        out_type=jax.ShapeDtypeStruct((B, L, OUT_W), jnp.float32),
        mesh=mesh,
        scratch_types=[
            pltpu.VMEM((BROWS,), jnp.int32),
            pltpu.VMEM((BROWS,), jnp.int32),
            pltpu.VMEM((BROWS,), jnp.float32),
            pltpu.VMEM((2 * TAB,), jnp.int32),
            pltpu.VMEM((CH, OUT_W), jnp.float32),
            pltpu.VMEM((CH, OUT_W), jnp.float32),
            pltpu.SemaphoreType.DMA,
        ],
        compiler_params=pltpu.CompilerParams(needs_layout_passes=False),
    )
    return run(de1f, de2f, ff, tab)


# speedup vs baseline: 1.2565x; 1.2565x over previous
import functools

import jax
import jax.numpy as jnp
from jax import lax
from jax.experimental import pallas as pl
from jax.experimental.pallas import tpu as pltpu
from jax.experimental.pallas import tpu_sc as plsc

E = 64
OUT_W = 2 * E + 1
PW = E // 2
STRIDE = PW + 1
TAB = 1000 * STRIDE
L_DIM = 200
CH = 80
BLK_B = 4
BROWS = BLK_B * L_DIM
NCH = BROWS // CH


def _chunk_dsts(u):
    l = CH * u
    b_off, l0 = divmod(l, L_DIM)
    if l0 + CH <= L_DIM:
        return [(b_off, l0, CH, 0)]
    s1 = L_DIM - l0
    return [(b_off, l0, s1, 0), (b_off + 1, 0, CH - s1, s1)]


def _sc_body(bs_per_w, de1_hbm, de2_hbm, f_hbm, tab_hbm, out_hbm,
             idx1_v, idx2_v, f_v, tab_v, sb_0, sb_1, sem_o):
    nc = 2
    wid = lax.axis_index("s") * nc + lax.axis_index("c")
    base_b = wid * bs_per_w
    lanes = lax.iota(jnp.int32, 16)
    sbs = (sb_0, sb_1)
    n_blk = bs_per_w // BLK_B

    pltpu.sync_copy(tab_hbm, tab_v)

    def out_copies(b0, u, p):
        res = []
        for b_off, l0, sz, loc in _chunk_dsts(u):
            res.append(pltpu.make_async_copy(
                sbs[p].at[pl.ds(loc, sz)],
                out_hbm.at[b0 + b_off, pl.ds(l0, sz)], sem_o))
        return res

    def assemble(u, p):
        sb = sbs[p]
        mask16 = jnp.full((16,), -65536, jnp.int32)
        col128 = lanes * 0 + (2 * E)
        for g in range(CH // 16):
            rows = lanes + 16 * g
            off = CH * u + 16 * g
            i1 = idx1_v[pl.ds(off, 16)]
            i2 = idx2_v[pl.ds(off, 16)]
            fv = f_v[pl.ds(off, 16)]
            ga1_0 = i1 * STRIDE
            ga2_0 = i2 * STRIDE + TAB
            plsc.store_scatter(sb, [rows, col128], fv)

            def wave(cb, carry):
                a1, a2 = carry
                v1s = [plsc.load_gather(tab_v, [a1 + d]) for d in range(4)]
                v2s = [plsc.load_gather(tab_v, [a2 + d]) for d in range(4)]
                e1s, e2s = [], []
                for d in range(4):
                    e1s.append(plsc.bitcast(v1s[d] << 16, jnp.float32))
                    e1s.append(plsc.bitcast(v1s[d] & mask16, jnp.float32))
                    e2s.append(plsc.bitcast(v2s[d] << 16, jnp.float32))
                    e2s.append(plsc.bitcast(v2s[d] & mask16, jnp.float32))
                cbase = cb * 8
                for d in range(8):
                    plsc.store_scatter(sb, [rows, cbase + d + (lanes * 0)],
                                       e1s[d])
                for d in range(8):
                    plsc.store_scatter(sb, [rows, cbase + d + E + (lanes * 0)],
                                       e2s[d])
                return a1 + 4, a2 + 4

            lax.fori_loop(0, PW // 4, wave, (ga1_0, ga2_0))

    def blk_body(m, _):
        b0 = base_b + m * BLK_B
        r0 = pl.multiple_of(b0 * L_DIM, BROWS)
        pltpu.sync_copy(de1_hbm.at[pl.ds(r0, BROWS)], idx1_v)
        pltpu.sync_copy(de2_hbm.at[pl.ds(r0, BROWS)], idx2_v)
        pltpu.sync_copy(f_hbm.at[pl.ds(r0, BROWS)], f_v)
        for u in range(NCH):
            p = u % 2
            if u >= 2:
                for c in out_copies(b0, u - 2, p):
                    c.wait()
            else:
                @pl.when(m > 0)
                def _wait_prev():
                    for c in out_copies(b0 - BLK_B, NCH - 2 + u, p):
                        c.wait()
            assemble(u, p)
            for c in out_copies(b0, u, p):
                c.start()

    lax.fori_loop(0, n_blk, blk_body, None)
    b_last = base_b + (n_blk - 1) * BLK_B
    for u in (NCH - 2, NCH - 1):
        for c in out_copies(b_last, u, u % 2):
            c.wait()


def kernel(de1, de2, f, W1, W2):
    B, L = de1.shape
    n = B * L
    info = plsc.get_sparse_core_info()
    nw = info.num_cores * info.num_subcores
    bs_per_w = B // nw
    assert bs_per_w % BLK_B == 0 and L == L_DIM

    de1f = de1.reshape(n)
    de2f = de2.reshape(n)
    ff = f.reshape(n)
    p1 = lax.bitcast_convert_type(
        W1.astype(jnp.bfloat16).reshape(1000, PW, 2), jnp.int32)
    p2 = lax.bitcast_convert_type(
        W2.astype(jnp.bfloat16).reshape(1000, PW, 2), jnp.int32)
    p1 = jnp.pad(p1, ((0, 0), (0, 1)))
    p2 = jnp.pad(p2, ((0, 0), (0, 1)))
    tab = jnp.concatenate([p1.reshape(-1), p2.reshape(-1)])

    mesh = plsc.VectorSubcoreMesh(core_axis_name="c", subcore_axis_name="s")
    run = pl.kernel(
        functools.partial(_sc_body, bs_per_w),
        out_type=jax.ShapeDtypeStruct((B, L, OUT_W), jnp.float32),
        mesh=mesh,
        scratch_types=[
            pltpu.VMEM((BROWS,), jnp.int32),
            pltpu.VMEM((BROWS,), jnp.int32),
            pltpu.VMEM((BROWS,), jnp.float32),
            pltpu.VMEM((2 * TAB,), jnp.int32),
            pltpu.VMEM((CH, OUT_W), jnp.float32),
            pltpu.VMEM((CH, OUT_W), jnp.float32),
            pltpu.SemaphoreType.DMA,
        ],
        compiler_params=pltpu.CompilerParams(needs_layout_passes=False),
    )
    return run(de1f, de2f, ff, tab)


# speedup vs baseline: 1.9808x; 1.5764x over previous
import functools

import jax
import jax.numpy as jnp
from jax import lax
from jax.experimental import pallas as pl
from jax.experimental.pallas import tpu as pltpu
from jax.experimental.pallas import tpu_sc as plsc

E = 64
OUT_W = 2 * E + 1
L_DIM = 200
CH = 80
BLK_B = 4
BROWS = BLK_B * L_DIM
NCH = BROWS // CH


def _chunk_dsts(u):
    l = CH * u
    b_off, l0 = divmod(l, L_DIM)
    if l0 + CH <= L_DIM:
        return [(b_off, l0, CH, 0)]
    s1 = L_DIM - l0
    return [(b_off, l0, s1, 0), (b_off + 1, 0, CH - s1, s1)]


def _sc_body(bs_per_w, de1_hbm, de2_hbm, f_hbm, w1_hbm, w2_hbm, out_hbm,
             idx1_v, idx2_v, f_v, g1_0, g1_1, g2_0, g2_1,
             sb_0, sb_1, fb_0, fb_1, sem1, sem2, sem_o):
    nc = 2
    wid = lax.axis_index("s") * nc + lax.axis_index("c")
    base_b = wid * bs_per_w
    lanes = lax.iota(jnp.int32, 16)
    g1s, g2s = (g1_0, g1_1), (g2_0, g2_1)
    sbs, fbs = (sb_0, sb_1), (fb_0, fb_1)
    n_blk = bs_per_w // BLK_B

    def gather(u, p):
        pltpu.async_copy(w1_hbm.at[idx1_v.at[pl.ds(CH * u, CH)]],
                         g1s[p], sem1)
        pltpu.async_copy(w2_hbm.at[idx2_v.at[pl.ds(CH * u, CH)]],
                         g2s[p], sem2)

    def gather_wait(u, p):
        pltpu.make_async_copy(w1_hbm.at[idx1_v.at[pl.ds(CH * u, CH)]],
                              g1s[p], sem1).wait()
        pltpu.make_async_copy(w2_hbm.at[idx2_v.at[pl.ds(CH * u, CH)]],
                              g2s[p], sem2).wait()

    def out_copies(b0, u, p):
        res = []
        for b_off, l0, sz, loc in _chunk_dsts(u):
            res.append(pltpu.make_async_copy(
                sbs[p].at[pl.ds(loc, sz)],
                out_hbm.at[b0 + b_off, pl.ds(l0, sz), pl.ds(0, 128)], sem_o))
            res.append(pltpu.make_async_copy(
                fbs[p].at[pl.ds(loc, sz)],
                out_hbm.at[b0 + b_off, pl.ds(l0, sz), pl.ds(128, 1)], sem_o))
        return res

    def assemble(u, p):
        g1, g2, sb, fb = g1s[p], g2s[p], sbs[p], fbs[p]

        def row_body(l, _):
            vals = ([g1[l, pl.ds(16 * k, 16)] for k in range(E // 16)] +
                    [g2[l, pl.ds(16 * k, 16)] for k in range(E // 16)])
            for k in range(2 * (E // 16)):
                sb[l, pl.ds(16 * k, 16)] = vals[k]

        lax.fori_loop(0, CH, row_body, None, unroll=2)
        zeros = lanes * 0
        for g in range(CH // 16):
            rows = lanes + 16 * g
            fv = f_v[pl.ds(CH * u + 16 * g, 16)]
            plsc.store_scatter(fb, [rows, zeros], fv)

    def blk_body(m, _):
        b0 = base_b + m * BLK_B
        r0 = pl.multiple_of(b0 * L_DIM, BROWS)
        pltpu.sync_copy(de1_hbm.at[pl.ds(r0, BROWS)], idx1_v)
        pltpu.sync_copy(de2_hbm.at[pl.ds(r0, BROWS)], idx2_v)
        pltpu.sync_copy(f_hbm.at[pl.ds(r0, BROWS)], f_v)
        gather(0, 0)
        for u in range(NCH):
            p = u % 2
            if u + 1 < NCH:
                gather(u + 1, 1 - p)
            if u >= 2:
                for c in out_copies(b0, u - 2, p):
                    c.wait()
            else:
                @pl.when(m > 0)
                def _wait_prev():
                    for c in out_copies(b0 - BLK_B, NCH - 2 + u, p):
                        c.wait()
            gather_wait(u, p)
            assemble(u, p)
            for c in out_copies(b0, u, p):
                c.start()

    lax.fori_loop(0, n_blk, blk_body, None)
    b_last = base_b + (n_blk - 1) * BLK_B
    for u in (NCH - 2, NCH - 1):
        for c in out_copies(b_last, u, u % 2):
            c.wait()


def kernel(de1, de2, f, W1, W2):
    B, L = de1.shape
    n = B * L
    info = plsc.get_sparse_core_info()
    nw = info.num_cores * info.num_subcores
    bs_per_w = B // nw
    assert bs_per_w % BLK_B == 0 and L == L_DIM

    de1f = de1.reshape(n)
    de2f = de2.reshape(n)
    ff = f.reshape(n)
    W1p = jnp.pad(W1, ((0, 0), (0, 128 - E)))
    W2p = jnp.pad(W2, ((0, 0), (0, 128 - E)))

    mesh = plsc.VectorSubcoreMesh(core_axis_name="c", subcore_axis_name="s")
    run = pl.kernel(
        functools.partial(_sc_body, bs_per_w),
        out_type=jax.ShapeDtypeStruct((B, L, OUT_W), jnp.float32),
        mesh=mesh,
        scratch_types=[
            pltpu.VMEM((BROWS,), jnp.int32),
            pltpu.VMEM((BROWS,), jnp.int32),
            pltpu.VMEM((BROWS,), jnp.float32),
            pltpu.VMEM((CH, 128), jnp.float32),
            pltpu.VMEM((CH, 128), jnp.float32),
            pltpu.VMEM((CH, 128), jnp.float32),
            pltpu.VMEM((CH, 128), jnp.float32),
            pltpu.VMEM((CH, 128), jnp.float32),
            pltpu.VMEM((CH, 128), jnp.float32),
            pltpu.VMEM((CH, 1), jnp.float32),
            pltpu.VMEM((CH, 1), jnp.float32),
            pltpu.SemaphoreType.DMA,
            pltpu.SemaphoreType.DMA,
            pltpu.SemaphoreType.DMA,
        ],
        compiler_params=pltpu.CompilerParams(needs_layout_passes=False),
    )
    return run(de1f, de2f, ff, W1p, W2p)
